# baseline (speedup 1.0000x reference)
import functools

import jax
import jax.numpy as jnp
from jax import lax
from jax.experimental import pallas as pl
from jax.experimental.pallas import tpu as pltpu
from jax.experimental.pallas import tpu_sc as plsc

_B = 16384
_D = 32
_NC = 2
_NS = 16
_NW = _NC * _NS
_BPW = _B // _NW
_CHUNK = 128
_NCHUNK = _BPW // _CHUNK


@functools.partial(
    pl.kernel,
    mesh=plsc.VectorSubcoreMesh(core_axis_name="c", subcore_axis_name="s"),
    out_type=jax.ShapeDtypeStruct((_NW, _BPW, _D), jnp.float32),
    scratch_types=[
        pltpu.VMEM((_NCHUNK, _CHUNK), jnp.int32),
        pltpu.VMEM((_BPW, _D), jnp.float32),
        pltpu.SemaphoreType.DMA,
    ],
    compiler_params=pltpu.CompilerParams(use_tc_tiling_on_sc=False),
)
def _sc_gather(idx_hbm, table_hbm, out_hbm, idx_v, rows_v, sem):
    wid = lax.axis_index("s") * _NC + lax.axis_index("c")
    pltpu.sync_copy(idx_hbm.at[wid], idx_v)
    copies = []
    for j in range(_NCHUNK):
        copies.append(
            pltpu.async_copy(
                table_hbm.at[idx_v.at[j]],
                rows_v.at[pl.ds(j * _CHUNK, _CHUNK)],
                sem,
            )
        )
    for c in copies:
        c.wait()
    pltpu.sync_copy(rows_v, out_hbm.at[wid])


def kernel(skills, table):
    idx = skills.astype(jnp.int32).reshape(_NW, _NCHUNK, _CHUNK)
    out = _sc_gather(idx, table)
    return out.reshape(_B, _D)


# speedup vs baseline: 1.0629x; 1.0629x over previous
import functools

import jax
import jax.numpy as jnp
from jax import lax
from jax.experimental import pallas as pl
from jax.experimental.pallas import tpu as pltpu
from jax.experimental.pallas import tpu_sc as plsc

_B = 16384
_D = 32
_NW = 32
_BPW = _B // _NW
_L = 16


@functools.partial(
    pl.kernel,
    mesh=plsc.VectorSubcoreMesh(core_axis_name="c", subcore_axis_name="s"),
    out_type=jax.ShapeDtypeStruct((_B, _D), jnp.float32),
    scratch_types=[
        pltpu.VMEM((_BPW,), jnp.int32),
        pltpu.VMEM((_BPW, _D), jnp.float32),
        pltpu.SemaphoreType.DMA,
    ],
    compiler_params=pltpu.CompilerParams(use_tc_tiling_on_sc=True),
)
def _probe(idx_hbm, table_hbm, out_hbm, idx_v, rows_v, sem):
    wid = lax.axis_index("s") * 2 + lax.axis_index("c")
    base = wid * _BPW
    pltpu.sync_copy(idx_hbm.at[pl.ds(base, _BPW)], idx_v)

    def body(g, _):
        k0 = g * _L
        vec = idx_v[pl.ds(k0, _L)]
        copies = []
        for j in range(_L):
            i = vec[j]
            copies.append(
                pltpu.async_copy(
                    table_hbm.at[pl.ds(i, 1), :],
                    rows_v.at[pl.ds(k0 + j, 1), :],
                    sem,
                )
            )
        for c in copies:
            c.wait()
        return ()

    lax.fori_loop(0, _BPW // _L, body, ())
    pltpu.sync_copy(rows_v, out_hbm.at[pl.ds(base, _BPW), :])


def kernel(skills, table):
    idx = skills.astype(jnp.int32)
    return _probe(idx, table)


# speedup vs baseline: 2.2315x; 2.0995x over previous
import functools

import jax
import jax.numpy as jnp
from jax import lax
from jax.experimental import pallas as pl
from jax.experimental.pallas import tpu as pltpu
from jax.experimental.pallas import tpu_sc as plsc

_B = 16384
_D = 32
_V = 100000
_NC = 2
_NS = 16
_NW = _NC * _NS
_L = 16
_CB = 4096
_NCHUNK = _B // _CB


@functools.partial(
    pl.kernel,
    mesh=plsc.VectorSubcoreMesh(core_axis_name="c", subcore_axis_name="s"),
    out_type=jax.ShapeDtypeStruct((_D, _B), jnp.float32),
    scratch_types=[
        pltpu.VMEM((_V,), jnp.float32),
        pltpu.VMEM((_CB,), jnp.int32),
        pltpu.VMEM((_CB,), jnp.float32),
        pltpu.SemaphoreType.DMA,
        pltpu.SemaphoreType.DMA,
    ],
    compiler_params=pltpu.CompilerParams(
        use_tc_tiling_on_sc=True, needs_layout_passes=False
    ),
)
def _sc_gather(idx_hbm, tab_hbm, out_hbm, col_v, idx_v, out_v, tsem, isem):
    wid = lax.axis_index("s") * _NC + lax.axis_index("c")
    tab_cp = pltpu.async_copy(tab_hbm.at[wid, :], col_v, tsem)

    def chunk(k, _):
        pltpu.async_copy(
            idx_hbm.at[pl.ds(k * _CB, _CB)], idx_v, isem
        ).wait()

        def group(g, _):
            iv = idx_v[pl.ds(g * _L, _L)]
            out_v[pl.ds(g * _L, _L)] = plsc.load_gather(col_v, [iv])
            return ()

        lax.fori_loop(0, _CB // _L, group, ())
        pltpu.sync_copy(out_v, out_hbm.at[wid, pl.ds(k * _CB, _CB)])
        return ()

    tab_cp.wait()
    lax.fori_loop(0, _NCHUNK, chunk, ())


def kernel(skills, table):
    idx = skills.astype(jnp.int32)
    out_t = _sc_gather(idx, table.T)
    return out_t.T


# speedup vs baseline: 2.5350x; 1.1360x over previous
import functools

import jax
import jax.numpy as jnp
from jax import lax
from jax.experimental import pallas as pl
from jax.experimental.pallas import tpu as pltpu
from jax.experimental.pallas import tpu_sc as plsc

_B = 16384
_D = 32
_V = 100000
_NC = 2
_NS = 16
_NW = _NC * _NS
_L = 16
_CB = 4096
_NCHUNK = _B // _CB
_UNROLL = 8


@functools.partial(
    pl.kernel,
    mesh=plsc.VectorSubcoreMesh(core_axis_name="c", subcore_axis_name="s"),
    out_type=jax.ShapeDtypeStruct((_D, _B), jnp.float32),
    scratch_types=[
        pltpu.VMEM((_V,), jnp.float32),
        pltpu.VMEM((2, _CB), jnp.int32),
        pltpu.VMEM((2, _CB), jnp.float32),
        pltpu.SemaphoreType.DMA,
        pltpu.SemaphoreType.DMA,
        pltpu.SemaphoreType.DMA,
    ],
    compiler_params=pltpu.CompilerParams(
        use_tc_tiling_on_sc=True, needs_layout_passes=False
    ),
)
def _sc_gather(idx_hbm, tab_hbm, out_hbm, col_v, idx_v, out_v, tsem, isem, osem):
    wid = lax.axis_index("s") * _NC + lax.axis_index("c")
    tab_cp = pltpu.async_copy(tab_hbm.at[wid, :], col_v, tsem)
    pltpu.async_copy(idx_hbm.at[pl.ds(0, _CB)], idx_v.at[0], isem)
    tab_cp.wait()

    def run_chunk(k, buf, nbuf):
        pltpu.make_async_copy(
            idx_hbm.at[pl.ds(0, _CB)], idx_v.at[buf], isem
        ).wait()
        if k + 1 < _NCHUNK:
            pltpu.async_copy(
                idx_hbm.at[pl.ds((k + 1) * _CB, _CB)], idx_v.at[nbuf], isem
            )
        if k >= 2:
            pltpu.make_async_copy(
                out_v.at[buf], out_hbm.at[wid, pl.ds(0, _CB)], osem
            ).wait()

        def group(g, _):
            for u in range(_UNROLL):
                o = g * (_L * _UNROLL) + u * _L
                iv = idx_v[buf, pl.ds(o, _L)]
                out_v[buf, pl.ds(o, _L)] = plsc.load_gather(col_v, [iv])
            return ()

        lax.fori_loop(0, _CB // (_L * _UNROLL), group, ())
        pltpu.async_copy(
            out_v.at[buf], out_hbm.at[wid, pl.ds(k * _CB, _CB)], osem
        )

    for k in range(_NCHUNK):
        run_chunk(k, k % 2, (k + 1) % 2)
    for k in range(_NCHUNK - 2, _NCHUNK):
        pltpu.make_async_copy(
            out_v.at[k % 2], out_hbm.at[wid, pl.ds(0, _CB)], osem
        ).wait()


def kernel(skills, table):
    idx = skills.astype(jnp.int32)
    out_t = _sc_gather(idx, table.T)
    return out_t.T


# speedup vs baseline: 2.6160x; 1.0320x over previous
import functools

import jax
import jax.numpy as jnp
from jax import lax
from jax.experimental import pallas as pl
from jax.experimental.pallas import tpu as pltpu
from jax.experimental.pallas import tpu_sc as plsc

_B = 16384
_D = 32
_V = 100000
_NC = 2
_NS = 16
_NW = _NC * _NS
_L = 16
_CB = 4096
_NCHUNK = _B // _CB
_UNROLL = 8


@functools.partial(
    pl.kernel,
    mesh=plsc.VectorSubcoreMesh(core_axis_name="c", subcore_axis_name="s"),
    out_type=jax.ShapeDtypeStruct((_D, _B), jnp.float32),
    scratch_types=[
        pltpu.VMEM((_V,), jnp.float32),
        pltpu.VMEM((2, _CB), jnp.int32),
        pltpu.VMEM((2, _CB), jnp.float32),
        pltpu.SemaphoreType.DMA,
        pltpu.SemaphoreType.DMA,
        pltpu.SemaphoreType.DMA,
    ],
    compiler_params=pltpu.CompilerParams(
        use_tc_tiling_on_sc=True, needs_layout_passes=False
    ),
)
def _sc_gather(idx_hbm, tab_hbm, out_hbm, col_v, idx_v, out_v, tsem, isem, osem):
    wid = lax.axis_index("c") * _NS + lax.axis_index("s")
    tab_cp = pltpu.async_copy(tab_hbm.at[wid, :], col_v, tsem)
    pltpu.async_copy(idx_hbm.at[pl.ds(0, _CB)], idx_v.at[0], isem)
    tab_cp.wait()

    def run_chunk(k, buf, nbuf):
        pltpu.make_async_copy(
            idx_hbm.at[pl.ds(0, _CB)], idx_v.at[buf], isem
        ).wait()
        if k + 1 < _NCHUNK:
            pltpu.async_copy(
                idx_hbm.at[pl.ds((k + 1) * _CB, _CB)], idx_v.at[nbuf], isem
            )
        if k >= 2:
            pltpu.make_async_copy(
                out_v.at[buf], out_hbm.at[wid, pl.ds(0, _CB)], osem
            ).wait()

        @plsc.parallel_loop(0, _CB, _L, unroll=_UNROLL)
        def _gather(o):
            iv = idx_v[buf, pl.ds(o, _L)]
            out_v[buf, pl.ds(o, _L)] = plsc.load_gather(col_v, [iv])
        pltpu.async_copy(
            out_v.at[buf], out_hbm.at[wid, pl.ds(k * _CB, _CB)], osem
        )

    for k in range(_NCHUNK):
        run_chunk(k, k % 2, (k + 1) % 2)
    for k in range(_NCHUNK - 2, _NCHUNK):
        pltpu.make_async_copy(
            out_v.at[k % 2], out_hbm.at[wid, pl.ds(0, _CB)], osem
        ).wait()


def kernel(skills, table):
    idx = skills.astype(jnp.int32)
    out_t = _sc_gather(idx, table.T)
    return out_t.T
